# baseline (speedup 1.0000x reference)
import functools

import jax
import jax.numpy as jnp
from jax import lax
from jax.experimental import pallas as pl
from jax.experimental.pallas import tpu as pltpu
from jax.experimental.pallas import tpu_sc as plsc

N_ENT = 50000
N_ITEM = 10000
D = 128
N_REL = 12
R = N_REL - 1
E_KG = 500000
E_II = 320000

NC = 2
NS = 16
NW = NC * NS

E_KGP = 512000
KG_CHUNK = 128
KG_GROUPS = 5
KG_GCHUNK = 25
KG_K = 4
II_CHUNK = 125
II_GROUPS = 4
II_GCHUNK = 20
II_K = 2

CB = 4
CW = 32
ENT_PAD = 50176
ITEM_PAD = 10240
ENT_ROWS_PER_TILE = ENT_PAD // NS
ITEM_ROWS_PER_TILE = ITEM_PAD // NS
ENT_ZCHUNK = 56
ITEM_ZCHUNK = 64
EC_ROWS_PER_TILE = ENT_PAD // NW
EC_CHUNK = 224

_mesh = plsc.VectorSubcoreMesh(core_axis_name="c", subcore_axis_name="s")
_sc_params = pltpu.CompilerParams(use_tc_tiling_on_sc=False,
                                  needs_layout_passes=False)


@functools.partial(
    pl.kernel,
    out_type=jax.ShapeDtypeStruct((CB, ENT_PAD, CW), jnp.float32),
    mesh=_mesh,
    compiler_params=_sc_params,
    scratch_types=[
        pltpu.VMEM((EC_CHUNK, D), jnp.float32),
        pltpu.VMEM((CB, EC_CHUNK, CW), jnp.float32),
    ],
)
def _ecols_pass(e_hbm, out_hbm, ebuf, obuf):
    c = lax.axis_index("c")
    s = lax.axis_index("s")
    wid = c * NS + s

    @pl.loop(0, EC_ROWS_PER_TILE // EC_CHUNK)
    def _(ch):
        base = wid * EC_ROWS_PER_TILE + ch * EC_CHUNK
        pltpu.sync_copy(e_hbm.at[pl.ds(base, EC_CHUNK)], ebuf)

        @pl.loop(0, EC_CHUNK)
        def _(i):
            for p in range(CB):
                for q in range(CW // 16):
                    obuf[p, i, pl.ds(q * 16, 16)] = (
                        ebuf[i, pl.ds(p * CW + q * 16, 16)])
        for p in range(CB):
            pltpu.sync_copy(obuf.at[p],
                            out_hbm.at[p].at[pl.ds(base, EC_CHUNK)])


def _modulate(vals, w_v, rel_v, b, j, p):
    iota = lax.iota(jnp.int32, 16)
    bsplat = jnp.full((16,), b, jnp.int32)

    @pl.loop(0, KG_CHUNK // 16)
    def _(k):
        base = k * 16
        rel16 = rel_v[j, pl.ds(base, 16)]
        rows = base + iota
        for cc in range(CW):
            csplat = jnp.full((16,), cc, jnp.int32)
            wv = plsc.load_gather(
                w_v, [rel16, jnp.full((16,), p * CW + cc, jnp.int32)])
            vv = plsc.load_gather(vals, [bsplat, rows, csplat])
            plsc.store_scatter(vals, [bsplat, rows, csplat], vv * wv)


@functools.partial(
    pl.kernel,
    out_type=jax.ShapeDtypeStruct((NC, CB, ENT_PAD, CW), jnp.float32),
    mesh=_mesh,
    compiler_params=_sc_params,
    scratch_types=[
        pltpu.VMEM((KG_GCHUNK, KG_CHUNK), jnp.int32),
        pltpu.VMEM((KG_GCHUNK, KG_CHUNK), jnp.int32),
        pltpu.VMEM((KG_GCHUNK, KG_CHUNK), jnp.int32),
        pltpu.VMEM((16, D), jnp.float32),
        pltpu.VMEM((KG_K, KG_CHUNK, CW), jnp.float32),
        pltpu.VMEM((ENT_ZCHUNK, CW), jnp.float32),
        pltpu.VMEM_SHARED((ENT_PAD, CW), jnp.float32),
        pltpu.SemaphoreType.DMA,
        pltpu.SemaphoreType.DMA,
    ],
)
def _kg_pass(ec_hbm, neigh_hbm, rel_hbm, dst_hbm, w_hbm, out_hbm,
             neigh_v, dst_v, rel_sm, w_v, vals, zbuf, acc, gsem, ssem):
    c = lax.axis_index("c")
    s = lax.axis_index("s")
    wid = c * NS + s

    pltpu.sync_copy(w_hbm, w_v)
    zero = jnp.zeros((16,), jnp.float32)

    @pl.loop(0, ENT_ZCHUNK)
    def _(i):
        zbuf[i, pl.ds(0, 16)] = zero
        zbuf[i, pl.ds(16, 16)] = zero

    for p in range(CB):
        @pl.loop(0, ENT_ROWS_PER_TILE // ENT_ZCHUNK)
        def _(j):
            pltpu.sync_copy(
                zbuf, acc.at[pl.ds(s * ENT_ROWS_PER_TILE + j * ENT_ZCHUNK,
                                   ENT_ZCHUNK)])
        plsc.subcore_barrier()

        @pl.loop(0, KG_GROUPS)
        def _(g):
            pltpu.sync_copy(neigh_hbm.at[wid].at[g], neigh_v)
            pltpu.sync_copy(dst_hbm.at[wid].at[g], dst_v)
            pltpu.sync_copy(rel_hbm.at[wid].at[g], rel_sm)

            @pl.loop(0, KG_GCHUNK // KG_K)
            def _(bb):
                gathers = []
                for b in range(KG_K):
                    gathers.append(pltpu.async_copy(
                        ec_hbm.at[p].at[neigh_v.at[bb * KG_K + b]],
                        vals.at[b], gsem))
                scatters = []
                for b in range(KG_K):
                    j = bb * KG_K + b
                    gathers[b].wait()
                    _modulate(vals, w_v, rel_sm, b, j, p)
                    scatters.append(pltpu.async_copy(
                        vals.at[b], acc.at[dst_v.at[j]], ssem, add=True))
                for b in range(KG_K):
                    scatters[b].wait()

            jt = KG_GCHUNK - 1
            pltpu.async_copy(ec_hbm.at[p].at[neigh_v.at[jt]], vals.at[0],
                             gsem).wait()
            _modulate(vals, w_v, rel_sm, 0, jt, p)
            pltpu.async_copy(vals.at[0], acc.at[dst_v.at[jt]], ssem,
                             add=True).wait()
        plsc.subcore_barrier()

        pltpu.sync_copy(
            acc.at[pl.ds(s * ENT_ROWS_PER_TILE, ENT_ROWS_PER_TILE)],
            out_hbm.at[c].at[p].at[pl.ds(s * ENT_ROWS_PER_TILE,
                                         ENT_ROWS_PER_TILE)])


@functools.partial(
    pl.kernel,
    out_type=jax.ShapeDtypeStruct((NC, ITEM_PAD, D), jnp.float32),
    mesh=_mesh,
    compiler_params=_sc_params,
    scratch_types=[
        pltpu.VMEM((II_GCHUNK, II_CHUNK), jnp.int32),
        pltpu.VMEM((II_GCHUNK, II_CHUNK), jnp.int32),
        pltpu.VMEM((II_K, II_CHUNK, D), jnp.float32),
        pltpu.VMEM((ITEM_ZCHUNK, D), jnp.float32),
        pltpu.VMEM_SHARED((ITEM_PAD, D), jnp.float32),
        pltpu.SemaphoreType.DMA,
        pltpu.SemaphoreType.DMA,
    ],
)
def _ii_pass(emb_hbm, src_hbm, dst_hbm, out_hbm, src_v, dst_v, vals, zbuf,
             acc, gsem, ssem):
    c = lax.axis_index("c")
    s = lax.axis_index("s")
    wid = c * NS + s

    zero = jnp.zeros((16,), jnp.float32)

    @pl.loop(0, ITEM_ZCHUNK)
    def _(i):
        for q in range(D // 16):
            zbuf[i, pl.ds(q * 16, 16)] = zero

    @pl.loop(0, ITEM_ROWS_PER_TILE // ITEM_ZCHUNK)
    def _(j):
        pltpu.sync_copy(
            zbuf, acc.at[pl.ds(s * ITEM_ROWS_PER_TILE + j * ITEM_ZCHUNK,
                               ITEM_ZCHUNK)])
    plsc.subcore_barrier()

    @pl.loop(0, II_GROUPS)
    def _(g):
        pltpu.sync_copy(src_hbm.at[wid].at[g], src_v)
        pltpu.sync_copy(dst_hbm.at[wid].at[g], dst_v)

        @pl.loop(0, II_GCHUNK // II_K)
        def _(bb):
            gathers = []
            for b in range(II_K):
                gathers.append(pltpu.async_copy(
                    emb_hbm.at[src_v.at[bb * II_K + b]], vals.at[b], gsem))
            scatters = []
            for b in range(II_K):
                gathers[b].wait()
                scatters.append(pltpu.async_copy(
                    vals.at[b], acc.at[dst_v.at[bb * II_K + b]], ssem,
                    add=True))
            for b in range(II_K):
                scatters[b].wait()
    plsc.subcore_barrier()

    pltpu.sync_copy(
        acc.at[pl.ds(s * ITEM_ROWS_PER_TILE, ITEM_ROWS_PER_TILE)],
        out_hbm.at[c].at[pl.ds(s * ITEM_ROWS_PER_TILE, ITEM_ROWS_PER_TILE)])


_EN_CHUNK = 3136


def _ent_norm_body(p_ref, o_ref):
    x = p_ref[0] + p_ref[1]
    cols = jnp.concatenate([x[p] for p in range(CB)], axis=-1)
    norm = jnp.sqrt(jnp.sum(cols * cols, axis=-1, keepdims=True))
    o_ref[...] = cols / jnp.maximum(norm, 1e-12)


def _ent_combine_norm(parts):
    nsteps = ENT_PAD // _EN_CHUNK
    return pl.pallas_call(
        _ent_norm_body,
        grid=(nsteps,),
        in_specs=[pl.BlockSpec((NC, CB, _EN_CHUNK, CW),
                               lambda n: (0, 0, n, 0))],
        out_specs=pl.BlockSpec((_EN_CHUNK, D), lambda n: (n, 0)),
        out_shape=jax.ShapeDtypeStruct((ENT_PAD, D), jnp.float32),
    )(parts)


_IN_CHUNK = 2000


def _item_norm_body(p_ref, o_ref):
    x = p_ref[0] + p_ref[1]
    norm = jnp.sqrt(jnp.sum(x * x, axis=-1, keepdims=True))
    o_ref[...] = x / jnp.maximum(norm, 1e-12)


def _item_combine_norm(parts):
    nsteps = N_ITEM // _IN_CHUNK
    return pl.pallas_call(
        _item_norm_body,
        grid=(nsteps,),
        in_specs=[pl.BlockSpec((NC, _IN_CHUNK, D), lambda n: (0, n, 0))],
        out_specs=pl.BlockSpec((_IN_CHUNK, D), lambda n: (n, 0)),
        out_shape=jax.ShapeDtypeStruct((N_ITEM, D), jnp.float32),
    )(parts)


def kernel(entity_emb, item_emb, kg_rel, kg_neigh, kg_dst, ii_src, ii_dst,
           weight):
    npad = E_KGP - E_KG
    kgn = jnp.pad(kg_neigh.astype(jnp.int32), (0, npad)).reshape(
        NW, KG_GROUPS, KG_GCHUNK, KG_CHUNK)
    kgr = jnp.pad(kg_rel.astype(jnp.int32), (0, npad)).reshape(
        NW, KG_GROUPS, KG_GCHUNK, KG_CHUNK)
    kgd = jnp.pad(kg_dst.astype(jnp.int32), (0, npad),
                  constant_values=ENT_PAD - 1).reshape(
        NW, KG_GROUPS, KG_GCHUNK, KG_CHUNK)
    iis = ii_src.astype(jnp.int32).reshape(NW, II_GROUPS, II_GCHUNK,
                                           II_CHUNK)
    iid = ii_dst.astype(jnp.int32).reshape(NW, II_GROUPS, II_GCHUNK,
                                           II_CHUNK)
    w16 = jnp.pad(weight, ((0, 16 - R), (0, 0)))

    ii_parts = _ii_pass(item_emb, iis, iid)[:, :N_ITEM, :]
    ia = _item_combine_norm(ii_parts)

    e_pad = jnp.pad(entity_emb, ((0, ENT_PAD - N_ENT), (0, 0)))
    ent_out = [entity_emb]
    for _ in range(2):
        ecols = _ecols_pass(e_pad)
        parts = _kg_pass(ecols, kgn, kgr, kgd, w16)
        e_pad = _ent_combine_norm(parts)
        ent_out.append(e_pad[:N_ENT])

    return (jnp.stack(ent_out), jnp.stack([item_emb, ia, ia]))


# speedup vs baseline: 2.8416x; 2.8416x over previous
import functools

import jax
import jax.numpy as jnp
from jax import lax
from jax.experimental import pallas as pl
from jax.experimental.pallas import tpu as pltpu
from jax.experimental.pallas import tpu_sc as plsc

N_ENT = 50000
N_ITEM = 10000
D = 128
N_REL = 12
R = N_REL - 1
E_KG = 500000
E_II = 320000

NC = 2
NS = 16
NW = NC * NS

E_KGP = 512000
KG_CHUNK = 128
KG_GROUPS = 5
KG_GCHUNK = 25
KG_K = 4
II_CHUNK = 125
II_GROUPS = 4
II_GCHUNK = 20
II_K = 2

CB = 4
CW = 32
ENT_PAD = 50176
ITEM_PAD = 10240
ENT_ROWS_PER_TILE = ENT_PAD // NS
ITEM_ROWS_PER_TILE = ITEM_PAD // NS
ENT_ZCHUNK = 56
ITEM_ZCHUNK = 64
EC_ROWS_PER_TILE = ENT_PAD // NW
EC_CHUNK = 224

_mesh = plsc.VectorSubcoreMesh(core_axis_name="c", subcore_axis_name="s")
_sc_params = pltpu.CompilerParams(use_tc_tiling_on_sc=False,
                                  needs_layout_passes=False)


@functools.partial(
    pl.kernel,
    out_type=jax.ShapeDtypeStruct((CB, ENT_PAD, CW), jnp.float32),
    mesh=_mesh,
    compiler_params=_sc_params,
    scratch_types=[
        pltpu.VMEM((EC_CHUNK, D), jnp.float32),
        pltpu.VMEM((CB, EC_CHUNK, CW), jnp.float32),
    ],
)
def _ecols_pass(e_hbm, out_hbm, ebuf, obuf):
    c = lax.axis_index("c")
    s = lax.axis_index("s")
    wid = c * NS + s

    @pl.loop(0, EC_ROWS_PER_TILE // EC_CHUNK)
    def _(ch):
        base = wid * EC_ROWS_PER_TILE + ch * EC_CHUNK
        pltpu.sync_copy(e_hbm.at[pl.ds(base, EC_CHUNK)], ebuf)

        @pl.loop(0, EC_CHUNK)
        def _(i):
            for p in range(CB):
                for q in range(CW // 16):
                    obuf[p, i, pl.ds(q * 16, 16)] = (
                        ebuf[i, pl.ds(p * CW + q * 16, 16)])
        for p in range(CB):
            pltpu.sync_copy(obuf.at[p],
                            out_hbm.at[p].at[pl.ds(base, EC_CHUNK)])


def _modulate(vals, w_v, rel_v, b, j, p):
    @pl.loop(0, KG_CHUNK // 16)
    def _(k):
        rel16 = rel_v[j, pl.ds(k * 16, 16)]
        for u in range(16):
            r = rel16[u]
            i = k * 16 + u
            for q in range(CW // 16):
                vals[b, i, pl.ds(q * 16, 16)] = (
                    vals[b, i, pl.ds(q * 16, 16)]
                    * w_v[r, pl.ds(p * CW + q * 16, 16)])


@functools.partial(
    pl.kernel,
    out_type=jax.ShapeDtypeStruct((NC, CB, ENT_PAD, CW), jnp.float32),
    mesh=_mesh,
    compiler_params=_sc_params,
    scratch_types=[
        pltpu.VMEM((KG_GCHUNK, KG_CHUNK), jnp.int32),
        pltpu.VMEM((KG_GCHUNK, KG_CHUNK), jnp.int32),
        pltpu.VMEM((KG_GCHUNK, KG_CHUNK), jnp.int32),
        pltpu.VMEM((16, D), jnp.float32),
        pltpu.VMEM((KG_K, KG_CHUNK, CW), jnp.float32),
        pltpu.VMEM((ENT_ZCHUNK, CW), jnp.float32),
        pltpu.VMEM_SHARED((ENT_PAD, CW), jnp.float32),
        pltpu.SemaphoreType.DMA,
        pltpu.SemaphoreType.DMA,
    ],
)
def _kg_pass(ec_hbm, neigh_hbm, rel_hbm, dst_hbm, w_hbm, out_hbm,
             neigh_v, dst_v, rel_sm, w_v, vals, zbuf, acc, gsem, ssem):
    c = lax.axis_index("c")
    s = lax.axis_index("s")
    wid = c * NS + s

    pltpu.sync_copy(w_hbm, w_v)
    zero = jnp.zeros((16,), jnp.float32)

    @pl.loop(0, ENT_ZCHUNK)
    def _(i):
        zbuf[i, pl.ds(0, 16)] = zero
        zbuf[i, pl.ds(16, 16)] = zero

    for p in range(CB):
        @pl.loop(0, ENT_ROWS_PER_TILE // ENT_ZCHUNK)
        def _(j):
            pltpu.sync_copy(
                zbuf, acc.at[pl.ds(s * ENT_ROWS_PER_TILE + j * ENT_ZCHUNK,
                                   ENT_ZCHUNK)])
        plsc.subcore_barrier()

        @pl.loop(0, KG_GROUPS)
        def _(g):
            pltpu.sync_copy(neigh_hbm.at[wid].at[g], neigh_v)
            pltpu.sync_copy(dst_hbm.at[wid].at[g], dst_v)
            pltpu.sync_copy(rel_hbm.at[wid].at[g], rel_sm)

            @pl.loop(0, KG_GCHUNK // KG_K)
            def _(bb):
                gathers = []
                for b in range(KG_K):
                    gathers.append(pltpu.async_copy(
                        ec_hbm.at[p].at[neigh_v.at[bb * KG_K + b]],
                        vals.at[b], gsem))
                scatters = []
                for b in range(KG_K):
                    j = bb * KG_K + b
                    gathers[b].wait()
                    _modulate(vals, w_v, rel_sm, b, j, p)
                    scatters.append(pltpu.async_copy(
                        vals.at[b], acc.at[dst_v.at[j]], ssem, add=True))
                for b in range(KG_K):
                    scatters[b].wait()

            jt = KG_GCHUNK - 1
            pltpu.async_copy(ec_hbm.at[p].at[neigh_v.at[jt]], vals.at[0],
                             gsem).wait()
            _modulate(vals, w_v, rel_sm, 0, jt, p)
            pltpu.async_copy(vals.at[0], acc.at[dst_v.at[jt]], ssem,
                             add=True).wait()
        plsc.subcore_barrier()

        pltpu.sync_copy(
            acc.at[pl.ds(s * ENT_ROWS_PER_TILE, ENT_ROWS_PER_TILE)],
            out_hbm.at[c].at[p].at[pl.ds(s * ENT_ROWS_PER_TILE,
                                         ENT_ROWS_PER_TILE)])


@functools.partial(
    pl.kernel,
    out_type=jax.ShapeDtypeStruct((NC, ITEM_PAD, D), jnp.float32),
    mesh=_mesh,
    compiler_params=_sc_params,
    scratch_types=[
        pltpu.VMEM((II_GCHUNK, II_CHUNK), jnp.int32),
        pltpu.VMEM((II_GCHUNK, II_CHUNK), jnp.int32),
        pltpu.VMEM((II_K, II_CHUNK, D), jnp.float32),
        pltpu.VMEM((ITEM_ZCHUNK, D), jnp.float32),
        pltpu.VMEM_SHARED((ITEM_PAD, D), jnp.float32),
        pltpu.SemaphoreType.DMA,
        pltpu.SemaphoreType.DMA,
    ],
)
def _ii_pass(emb_hbm, src_hbm, dst_hbm, out_hbm, src_v, dst_v, vals, zbuf,
             acc, gsem, ssem):
    c = lax.axis_index("c")
    s = lax.axis_index("s")
    wid = c * NS + s

    zero = jnp.zeros((16,), jnp.float32)

    @pl.loop(0, ITEM_ZCHUNK)
    def _(i):
        for q in range(D // 16):
            zbuf[i, pl.ds(q * 16, 16)] = zero

    @pl.loop(0, ITEM_ROWS_PER_TILE // ITEM_ZCHUNK)
    def _(j):
        pltpu.sync_copy(
            zbuf, acc.at[pl.ds(s * ITEM_ROWS_PER_TILE + j * ITEM_ZCHUNK,
                               ITEM_ZCHUNK)])
    plsc.subcore_barrier()

    @pl.loop(0, II_GROUPS)
    def _(g):
        pltpu.sync_copy(src_hbm.at[wid].at[g], src_v)
        pltpu.sync_copy(dst_hbm.at[wid].at[g], dst_v)

        @pl.loop(0, II_GCHUNK // II_K)
        def _(bb):
            gathers = []
            for b in range(II_K):
                gathers.append(pltpu.async_copy(
                    emb_hbm.at[src_v.at[bb * II_K + b]], vals.at[b], gsem))
            scatters = []
            for b in range(II_K):
                gathers[b].wait()
                scatters.append(pltpu.async_copy(
                    vals.at[b], acc.at[dst_v.at[bb * II_K + b]], ssem,
                    add=True))
            for b in range(II_K):
                scatters[b].wait()
    plsc.subcore_barrier()

    pltpu.sync_copy(
        acc.at[pl.ds(s * ITEM_ROWS_PER_TILE, ITEM_ROWS_PER_TILE)],
        out_hbm.at[c].at[pl.ds(s * ITEM_ROWS_PER_TILE, ITEM_ROWS_PER_TILE)])


_EN_CHUNK = 3136


def _ent_norm_body(p_ref, o_ref):
    x = p_ref[0] + p_ref[1]
    cols = jnp.concatenate([x[p] for p in range(CB)], axis=-1)
    norm = jnp.sqrt(jnp.sum(cols * cols, axis=-1, keepdims=True))
    o_ref[...] = cols / jnp.maximum(norm, 1e-12)


def _ent_combine_norm(parts):
    nsteps = ENT_PAD // _EN_CHUNK
    return pl.pallas_call(
        _ent_norm_body,
        grid=(nsteps,),
        in_specs=[pl.BlockSpec((NC, CB, _EN_CHUNK, CW),
                               lambda n: (0, 0, n, 0))],
        out_specs=pl.BlockSpec((_EN_CHUNK, D), lambda n: (n, 0)),
        out_shape=jax.ShapeDtypeStruct((ENT_PAD, D), jnp.float32),
    )(parts)


_IN_CHUNK = 2000


def _item_norm_body(p_ref, o_ref):
    x = p_ref[0] + p_ref[1]
    norm = jnp.sqrt(jnp.sum(x * x, axis=-1, keepdims=True))
    o_ref[...] = x / jnp.maximum(norm, 1e-12)


def _item_combine_norm(parts):
    nsteps = N_ITEM // _IN_CHUNK
    return pl.pallas_call(
        _item_norm_body,
        grid=(nsteps,),
        in_specs=[pl.BlockSpec((NC, _IN_CHUNK, D), lambda n: (0, n, 0))],
        out_specs=pl.BlockSpec((_IN_CHUNK, D), lambda n: (n, 0)),
        out_shape=jax.ShapeDtypeStruct((N_ITEM, D), jnp.float32),
    )(parts)


def kernel(entity_emb, item_emb, kg_rel, kg_neigh, kg_dst, ii_src, ii_dst,
           weight):
    npad = E_KGP - E_KG
    kgn = jnp.pad(kg_neigh.astype(jnp.int32), (0, npad)).reshape(
        NW, KG_GROUPS, KG_GCHUNK, KG_CHUNK)
    kgr = jnp.pad(kg_rel.astype(jnp.int32), (0, npad)).reshape(
        NW, KG_GROUPS, KG_GCHUNK, KG_CHUNK)
    kgd = jnp.pad(kg_dst.astype(jnp.int32), (0, npad),
                  constant_values=ENT_PAD - 1).reshape(
        NW, KG_GROUPS, KG_GCHUNK, KG_CHUNK)
    iis = ii_src.astype(jnp.int32).reshape(NW, II_GROUPS, II_GCHUNK,
                                           II_CHUNK)
    iid = ii_dst.astype(jnp.int32).reshape(NW, II_GROUPS, II_GCHUNK,
                                           II_CHUNK)
    w16 = jnp.pad(weight, ((0, 16 - R), (0, 0)))

    ii_parts = _ii_pass(item_emb, iis, iid)[:, :N_ITEM, :]
    ia = _item_combine_norm(ii_parts)

    e_pad = jnp.pad(entity_emb, ((0, ENT_PAD - N_ENT), (0, 0)))
    ent_out = [entity_emb]
    for _ in range(2):
        ecols = _ecols_pass(e_pad)
        parts = _kg_pass(ecols, kgn, kgr, kgd, w16)
        e_pad = _ent_combine_norm(parts)
        ent_out.append(e_pad[:N_ENT])

    return (jnp.stack(ent_out), jnp.stack([item_emb, ia, ia]))
